# baseline (speedup 1.0000x reference)
import jax
import jax.numpy as jnp
from jax.experimental import pallas as pl
from jax.experimental.pallas import tpu as pltpu

N_REAL = 20064
ROWS = 160
NP = ROWS * 128


def _iou_body(gt_ref, px1, py1, px2, py2, mo_ref, ga_ref, fg_ref, bg_ref):
    x1 = px1[...]
    y1 = py1[...]
    x2 = px2[...]
    y2 = py2[...]
    area = (x2 - x1 + 1.0) * (y2 - y1 + 1.0)

    best0 = jnp.full((ROWS, 128), -1.0, jnp.float32)
    bestg0 = jnp.zeros((ROWS, 128), jnp.int32)

    def body(g, carry):
        best, bestg = carry
        gx1 = gt_ref[g, 0]
        gy1 = gt_ref[g, 1]
        gx2 = gt_ref[g, 2]
        gy2 = gt_ref[g, 3]
        ab = (gx2 - gx1 + 1.0) * (gy2 - gy1 + 1.0)
        iw = jnp.maximum(jnp.minimum(x2, gx2) - jnp.maximum(x1, gx1) + 1.0, 0.0)
        ih = jnp.maximum(jnp.minimum(y2, gy2) - jnp.maximum(y1, gy1) + 1.0, 0.0)
        inter = iw * ih
        union = (area + ab) - inter
        iou = inter / jnp.maximum(union, 1e-8)
        upd = iou > best
        return jnp.maximum(best, iou), jnp.where(upd, g, bestg)

    best, bestg = jax.lax.fori_loop(0, 64, body, (best0, bestg0))

    r = jax.lax.broadcasted_iota(jnp.int32, (ROWS, 128), 0)
    c = jax.lax.broadcasted_iota(jnp.int32, (ROWS, 128), 1)
    flat = r * 128 + c
    real = flat < N_REAL

    fg = jnp.where(best >= 0.7, best, -1.0)
    bg = jnp.where((best < 0.5) & (best >= 0.1), best, -1.0)

    mo_ref[...] = best
    ga_ref[...] = bestg
    fg_ref[...] = jnp.where(real, fg, -2.0)
    bg_ref[...] = jnp.where(real, bg, -2.0)


def _iou_stage(px1, py1, px2, py2, gt):
    out = pl.pallas_call(
        _iou_body,
        out_shape=(
            jax.ShapeDtypeStruct((ROWS, 128), jnp.float32),
            jax.ShapeDtypeStruct((ROWS, 128), jnp.int32),
            jax.ShapeDtypeStruct((ROWS, 128), jnp.float32),
            jax.ShapeDtypeStruct((ROWS, 128), jnp.float32),
        ),
        in_specs=[
            pl.BlockSpec(memory_space=pltpu.SMEM),
            pl.BlockSpec(memory_space=pltpu.VMEM),
            pl.BlockSpec(memory_space=pltpu.VMEM),
            pl.BlockSpec(memory_space=pltpu.VMEM),
            pl.BlockSpec(memory_space=pltpu.VMEM),
        ],
    )(gt, px1, py1, px2, py2)
    return out


def _bbox_transform(ex, gt):
    ex_w = ex[:, 2] - ex[:, 0] + 1.0
    ex_h = ex[:, 3] - ex[:, 1] + 1.0
    ex_cx = ex[:, 0] + 0.5 * ex_w
    ex_cy = ex[:, 1] + 0.5 * ex_h
    gt_w = gt[:, 2] - gt[:, 0] + 1.0
    gt_h = gt[:, 3] - gt[:, 1] + 1.0
    gt_cx = gt[:, 0] + 0.5 * gt_w
    gt_cy = gt[:, 1] + 0.5 * gt_h
    dx = (gt_cx - ex_cx) / ex_w
    dy = (gt_cy - ex_cy) / ex_h
    dw = jnp.log(gt_w / ex_w)
    dh = jnp.log(gt_h / ex_h)
    return jnp.stack([dx, dy, dw, dh], axis=1)


def kernel(proposals, bounding_boxes, labels):
    props = jnp.concatenate([proposals, bounding_boxes], axis=1)[0]
    gt = bounding_boxes[0]
    lab = labels[0]

    pt = jnp.pad(props.T, ((0, 0), (0, NP - N_REAL)))
    pr = pt.reshape(4, ROWS, 128)

    mo, ga, fg, bg = _iou_stage(pr[0], pr[1], pr[2], pr[3], gt)

    mo_f = mo.reshape(-1)[:N_REAL]
    ga_f = ga.reshape(-1)[:N_REAL]
    fg_f = fg.reshape(-1)[:N_REAL]
    bg_f = bg.reshape(-1)[:N_REAL]

    _, fg_inds = jax.lax.top_k(fg_f, 64)
    _, bg_inds = jax.lax.top_k(bg_f, 64)
    keep_inds = jnp.concatenate([fg_inds, bg_inds], axis=0)

    rois = jnp.take(props, keep_inds, axis=0)
    prop_labels = jnp.take(lab, ga_f, axis=0)
    labels_keep = jnp.take(prop_labels, keep_inds, axis=0)
    is_fg = jnp.arange(128) < 64
    background = jax.nn.one_hot(0, lab.shape[1], dtype=labels_keep.dtype)
    labels_out = jnp.where(is_fg[:, None], labels_keep, background[None, :])
    gt_keep = jnp.take(gt, jnp.take(ga_f, keep_inds), axis=0)
    bbox_targets = _bbox_transform(rois, gt_keep)
    bbox_targets = jnp.where(is_fg[:, None], bbox_targets, 0.0)
    return (rois[None], labels_out[None], bbox_targets[None])


# speedup vs baseline: 3.6190x; 3.6190x over previous
import jax
import jax.numpy as jnp
from jax.experimental import pallas as pl
from jax.experimental.pallas import tpu as pltpu

N_REAL = 20064
ROWS = 160
NP = ROWS * 128
K = 64
F32 = jnp.float32


def _fused_body(gt_ref, px1, py1, px2, py2, gtv_ref, labv_ref,
                rois_ref, lab_ref, bbox_ref):
    x1 = px1[...]
    y1 = py1[...]
    x2 = px2[...]
    y2 = py2[...]
    area = (x2 - x1 + 1.0) * (y2 - y1 + 1.0)

    best0 = jnp.full((ROWS, 128), -1.0, F32)
    bestg0 = jnp.zeros((ROWS, 128), jnp.int32)

    def iou_step(g, carry):
        best, bestg = carry
        gx1 = gt_ref[g, 0]
        gy1 = gt_ref[g, 1]
        gx2 = gt_ref[g, 2]
        gy2 = gt_ref[g, 3]
        ab = (gx2 - gx1 + 1.0) * (gy2 - gy1 + 1.0)
        iw = jnp.maximum(jnp.minimum(x2, gx2) - jnp.maximum(x1, gx1) + 1.0, 0.0)
        ih = jnp.maximum(jnp.minimum(y2, gy2) - jnp.maximum(y1, gy1) + 1.0, 0.0)
        inter = iw * ih
        union = (area + ab) - inter
        iou = inter / jnp.maximum(union, 1e-8)
        upd = iou > best
        return jnp.maximum(best, iou), jnp.where(upd, g, bestg)

    best, bestg = jax.lax.fori_loop(0, 64, iou_step, (best0, bestg0))

    r_i = jax.lax.broadcasted_iota(jnp.int32, (ROWS, 128), 0)
    c_i = jax.lax.broadcasted_iota(jnp.int32, (ROWS, 128), 1)
    flat_f = (r_i * 128 + c_i).astype(F32)
    real = flat_f < float(N_REAL)

    sf0 = jnp.where(real & (best >= 0.7), best, jnp.where(real, -1.0, -2.0))
    sb0 = jnp.where(real & (best < 0.5) & (best >= 0.1), best,
                    jnp.where(real, -1.0, -2.0))

    lane128 = jax.lax.broadcasted_iota(jnp.int32, (1, 128), 1)
    sub128 = jax.lax.broadcasted_iota(jnp.int32, (128, 1), 0)
    BIG = jnp.float32(1e9)

    def sel_step(j, carry):
        sf, sb, krow, kcol = carry
        vf = jnp.max(sf)
        pf = jnp.min(jnp.where(sf == vf, flat_f, BIG))
        sf = jnp.where(flat_f == pf, -3.0, sf)
        vb = jnp.max(sb)
        pb = jnp.min(jnp.where(sb == vb, flat_f, BIG))
        sb = jnp.where(flat_f == pb, -3.0, sb)
        krow = jnp.where(lane128 == j, pf, krow)
        krow = jnp.where(lane128 == j + K, pb, krow)
        kcol = jnp.where(sub128 == j, pf, kcol)
        kcol = jnp.where(sub128 == j + K, pb, kcol)
        return sf, sb, krow, kcol

    krow0 = jnp.zeros((1, 128), F32)
    kcol0 = jnp.zeros((128, 1), F32)
    _, _, krow, kcol = jax.lax.fori_loop(
        0, K, sel_step, (sf0, sb0, krow0, kcol0))

    keep_i = kcol.astype(jnp.int32)
    keep_div = keep_i >> 7
    keep_mod = keep_i & 127

    ch = (jax.lax.broadcasted_iota(jnp.int32, (128, 128), 1)
          == keep_mod).astype(F32)
    rh = (jax.lax.broadcasted_iota(jnp.int32, (128, ROWS), 1)
          == keep_div).astype(F32)

    dn = (((1,), (1,)), ((), ()))

    def take(p2d):
        s = jax.lax.dot_general(ch, p2d, dn,
                                preferred_element_type=F32,
                                precision=jax.lax.Precision.HIGHEST)
        return jnp.sum(rh * s, axis=1, keepdims=True)

    rx1 = take(x1)
    ry1 = take(y1)
    rx2 = take(x2)
    ry2 = take(y2)
    ga_keep = take(bestg.astype(F32)).astype(jnp.int32)

    b2 = (jax.lax.broadcasted_iota(jnp.int32, (128, 64), 1)
          == ga_keep).astype(F32)
    labels_keep = jax.lax.dot_general(
        b2, labv_ref[...], (((1,), (0,)), ((), ())),
        preferred_element_type=F32, precision=jax.lax.Precision.HIGHEST)
    gt_keep = jax.lax.dot_general(
        b2, gtv_ref[...], (((1,), (0,)), ((), ())),
        preferred_element_type=F32, precision=jax.lax.Precision.HIGHEST)

    gx1 = gt_keep[:, 0:1]
    gy1 = gt_keep[:, 1:2]
    gx2 = gt_keep[:, 2:3]
    gy2 = gt_keep[:, 3:4]

    ex_w = rx2 - rx1 + 1.0
    ex_h = ry2 - ry1 + 1.0
    ex_cx = rx1 + 0.5 * ex_w
    ex_cy = ry1 + 0.5 * ex_h
    gt_w = gx2 - gx1 + 1.0
    gt_h = gy2 - gy1 + 1.0
    gt_cx = gx1 + 0.5 * gt_w
    gt_cy = gy1 + 0.5 * gt_h
    dx = (gt_cx - ex_cx) / ex_w
    dy = (gt_cy - ex_cy) / ex_h
    dw = jnp.log(gt_w / ex_w)
    dh = jnp.log(gt_h / ex_h)

    is_fg = (sub128 < K)
    lane_eq = lambda d: (lane128 == d).astype(F32)

    rois_ref[...] = (rx1 * lane_eq(0) + ry1 * lane_eq(1)
                     + rx2 * lane_eq(2) + ry2 * lane_eq(3))
    bbox = (dx * lane_eq(0) + dy * lane_eq(1)
            + dw * lane_eq(2) + dh * lane_eq(3))
    bbox_ref[...] = jnp.where(is_fg, bbox, 0.0)
    background = lane_eq(0)
    lab_ref[...] = jnp.where(is_fg, labels_keep,
                             jnp.broadcast_to(background, (128, 128)))


def kernel(proposals, bounding_boxes, labels):
    props = jnp.concatenate([proposals, bounding_boxes], axis=1)[0]
    gt = bounding_boxes[0]
    lab = labels[0]

    pt = jnp.pad(props.T, ((0, 0), (0, NP - N_REAL)))
    pr = pt.reshape(4, ROWS, 128)
    gtv = jnp.pad(gt, ((0, 0), (0, 124)))
    labv = jnp.pad(lab, ((0, 0), (0, 128 - lab.shape[1])))

    rois_p, lab_p, bbox_p = pl.pallas_call(
        _fused_body,
        out_shape=(
            jax.ShapeDtypeStruct((128, 128), F32),
            jax.ShapeDtypeStruct((128, 128), F32),
            jax.ShapeDtypeStruct((128, 128), F32),
        ),
        in_specs=[
            pl.BlockSpec(memory_space=pltpu.SMEM),
            pl.BlockSpec(memory_space=pltpu.VMEM),
            pl.BlockSpec(memory_space=pltpu.VMEM),
            pl.BlockSpec(memory_space=pltpu.VMEM),
            pl.BlockSpec(memory_space=pltpu.VMEM),
            pl.BlockSpec(memory_space=pltpu.VMEM),
            pl.BlockSpec(memory_space=pltpu.VMEM),
        ],
    )(gt, pr[0], pr[1], pr[2], pr[3], gtv, labv)

    rois = rois_p[:, :4]
    labels_out = lab_p[:, :lab.shape[1]]
    bbox_targets = bbox_p[:, :4]
    return (rois[None], labels_out[None], bbox_targets[None])
